# baseline (speedup 1.0000x reference)
import jax
import jax.numpy as jnp
from jax.experimental import pallas as pl

_NUM_NEIGHBORS = 8
_BETA = 0.5
_MIN_MARGIN = 0.05
_NUM_HARD_NEG = 4


def _loss_body(s_ref, t_ref, e_ref, f_ref, o_ref):
    N = s_ref.shape[1]

    def _norm(x):
        inv = 1.0 / jnp.maximum(
            jnp.sqrt(jnp.sum(x * x, axis=1, keepdims=True)), 1e-12)
        return x * inv

    s_n = _norm(s_ref[0])
    t_n = _norm(t_ref[0])

    dn = (((1,), (1,)), ((), ()))
    ssim = jax.lax.dot_general(s_n, s_n, dn, preferred_element_type=jnp.float32)
    tsim = jax.lax.dot_general(t_n, t_n, dn, preferred_element_type=jnp.float32)

    d2 = jax.lax.dot_general(e_ref[0], f_ref[0], dn,
                             preferred_element_type=jnp.float32)

    inf = jnp.float32(jnp.inf)

    m = jnp.min(d2, axis=1, keepdims=True)
    m1 = m
    for _ in range(_NUM_NEIGHBORS):
        m = jnp.min(jnp.where(d2 > m, d2, inf), axis=1, keepdims=True)
    knn = d2 <= m
    pos_mask = knn & (d2 > m1)

    neg = jnp.where(knn, -inf, tsim)
    g = jnp.max(neg, axis=1, keepdims=True)
    for _ in range(_NUM_HARD_NEG - 1):
        g = jnp.max(jnp.where(neg < g, neg, -inf), axis=1, keepdims=True)
    neg_mask = neg >= g

    zero = jnp.float32(0.0)
    inv_p = jnp.float32(1.0 / _NUM_NEIGHBORS)
    inv_n = jnp.float32(1.0 / _NUM_HARD_NEG)

    d = jnp.abs(jnp.where(pos_mask, ssim - tsim, zero))
    sl1 = jnp.where(d < _BETA, d * d, d - 0.5 * _BETA)
    pos_loss = jnp.sum(sl1, axis=1, keepdims=True) * inv_p

    w = jnp.where(pos_mask, inv_p, jnp.where(neg_mask, -inv_n, zero))
    dn2 = (((1,), (0,)), ((), ()))
    ws = jax.lax.dot_general(w, s_n, dn2, preferred_element_type=jnp.float32)
    wt = jax.lax.dot_general(w, t_n, dn2, preferred_element_type=jnp.float32)
    s_margin = jnp.sum(s_n * ws, axis=1, keepdims=True)
    t_margin = jnp.sum(t_n * wt, axis=1, keepdims=True)

    target = jnp.maximum(t_margin, jnp.float32(_MIN_MARGIN))
    margin_loss = jnp.maximum(target - s_margin, zero)
    per_patch = pos_loss + margin_loss
    o_ref[0, 0, :] = per_patch.reshape((N,))


def kernel(student_emb, teacher_emb, centers):
    B, N, D = student_emb.shape
    c2 = jnp.sum(centers * centers, axis=-1, keepdims=True)
    ones = jnp.ones_like(c2)
    pad = jnp.zeros((B, N, 3), dtype=centers.dtype)
    e = jnp.concatenate([centers, c2, ones, pad], axis=-1)
    f = jnp.concatenate([-2.0 * centers, ones, c2, pad], axis=-1)
    per_patch = pl.pallas_call(
        _loss_body,
        grid=(B,),
        in_specs=[
            pl.BlockSpec((1, N, D), lambda b: (b, 0, 0)),
            pl.BlockSpec((1, N, D), lambda b: (b, 0, 0)),
            pl.BlockSpec((1, N, 8), lambda b: (b, 0, 0)),
            pl.BlockSpec((1, N, 8), lambda b: (b, 0, 0)),
        ],
        out_specs=pl.BlockSpec((1, 1, N), lambda b: (b, 0, 0)),
        out_shape=jax.ShapeDtypeStruct((B, 1, N), jnp.float32),
    )(student_emb, teacher_emb, e, f)
    return per_patch.mean()


# speedup vs baseline: 1.2305x; 1.2305x over previous
import jax
import jax.numpy as jnp
from jax.experimental import pallas as pl

_NUM_NEIGHBORS = 8
_BETA = 0.5
_MIN_MARGIN = 0.05
_NUM_HARD_NEG = 4


def _loss_body(s_ref, t_ref, c_ref, o_ref):
    N = s_ref.shape[1]

    def _norm(x):
        inv = 1.0 / jnp.maximum(
            jnp.sqrt(jnp.sum(x * x, axis=1, keepdims=True)), 1e-12)
        return x * inv

    s_n = _norm(s_ref[0])
    t_n = _norm(t_ref[0])
    c_all = c_ref[0]

    dn = (((1,), (1,)), ((), ()))
    ssim = jax.lax.dot_general(s_n, s_n, dn, preferred_element_type=jnp.float32)
    tsim = jax.lax.dot_general(t_n, t_n, dn, preferred_element_type=jnp.float32)

    cross = jax.lax.dot_general(c_all, c_all, dn, preferred_element_type=jnp.float32)
    c2 = jnp.sum(c_all * c_all, axis=1, keepdims=True)
    d2 = c2 + jnp.transpose(c2) - 2.0 * cross

    inf = jnp.float32(jnp.inf)

    m = jnp.min(d2, axis=1, keepdims=True)
    m1 = m
    for _ in range(_NUM_NEIGHBORS):
        m = jnp.min(jnp.where(d2 > m, d2, inf), axis=1, keepdims=True)
    knn = d2 <= m
    pos_mask = knn & (d2 > m1)

    neg = jnp.where(knn, -inf, tsim)
    g = jnp.max(neg, axis=1, keepdims=True)
    for _ in range(_NUM_HARD_NEG - 1):
        g = jnp.max(jnp.where(neg < g, neg, -inf), axis=1, keepdims=True)
    neg_mask = neg >= g

    zero = jnp.float32(0.0)
    inv_p = jnp.float32(1.0 / _NUM_NEIGHBORS)
    inv_n = jnp.float32(1.0 / _NUM_HARD_NEG)

    d = jnp.abs(jnp.where(pos_mask, ssim - tsim, zero))
    sl1 = jnp.where(d < _BETA, d * d, d - 0.5 * _BETA)

    pos_loss = jnp.sum(sl1, axis=1, keepdims=True) * inv_p

    w = jnp.where(pos_mask, inv_p, jnp.where(neg_mask, -inv_n, zero))
    s_margin = jnp.sum(w * ssim, axis=1, keepdims=True)
    t_margin = jnp.sum(w * tsim, axis=1, keepdims=True)

    target = jnp.maximum(t_margin, jnp.float32(_MIN_MARGIN))
    margin_loss = jnp.maximum(target - s_margin, zero)
    per_patch = pos_loss + margin_loss
    o_ref[0, 0, :] = per_patch.reshape((N,))


def kernel(student_emb, teacher_emb, centers):
    B, N, D = student_emb.shape
    c_pad = jnp.pad(centers, ((0, 0), (0, 0), (0, 8 - centers.shape[-1])))
    per_patch = pl.pallas_call(
        _loss_body,
        grid=(B,),
        in_specs=[
            pl.BlockSpec((1, N, D), lambda b: (b, 0, 0)),
            pl.BlockSpec((1, N, D), lambda b: (b, 0, 0)),
            pl.BlockSpec((1, N, 8), lambda b: (b, 0, 0)),
        ],
        out_specs=pl.BlockSpec((1, 1, N), lambda b: (b, 0, 0)),
        out_shape=jax.ShapeDtypeStruct((B, 1, N), jnp.float32),
    )(student_emb, teacher_emb, c_pad)
    return per_patch.mean()
